# baseline (speedup 1.0000x reference)
import jax
import jax.numpy as jnp
from jax.experimental import pallas as pl

N = 10000
G = 16
L = 3


def _gat_conv(h_in, src, dst, W, asrc, adst, b):
    h = h_in @ W
    al_s = h @ asrc
    al_d = h @ adst
    e = jax.nn.leaky_relu(al_s[src] + al_d[dst], negative_slope=0.2)
    m = jax.ops.segment_max(e, dst, num_segments=N)
    m = jnp.where(jnp.isfinite(m), m, 0.0)
    ex = jnp.exp(e - m[dst])
    s = jax.ops.segment_sum(ex, dst, num_segments=N)
    alpha = ex / (s[dst] + 1e-16)
    out = jax.ops.segment_sum(h[src] * alpha[:, None], dst, num_segments=N)
    return out + b


def kernel(x, edge_index, batch, embed, Ws, a_src, a_dst, bs, W_out, b_out):
    src = edge_index[0]
    dst = edge_index[1]
    h = embed[x[:, 0]]
    for i in range(L - 1):
        h = jax.nn.relu(_gat_conv(h, src, dst, Ws[i], a_src[i], a_dst[i], bs[i]))
    h = _gat_conv(h, src, dst, Ws[L - 1], a_src[L - 1], a_dst[L - 1], bs[L - 1])
    sums = jax.ops.segment_sum(h, batch, num_segments=G)
    cnt = jax.ops.segment_sum(jnp.ones((N,), dtype=h.dtype), batch, num_segments=G)
    pooled = sums / jnp.clip(cnt, 1.0)[:, None]
    return pooled @ W_out + b_out
